# Initial kernel scaffold; baseline (speedup 1.0000x reference)
import jax
import jax.numpy as jnp
from jax.experimental import pallas as pl


def kernel(h_mf_new, h_bp_new, h_cc_new, mf_edge_index, bp_edge_index, cc_edge_index, W_mf, b_mf, W_bp, b_bp, W_cc, b_cc):
    raise NotImplementedError("write your pallas kernel here")


# speedup vs baseline: 3.3064x; 3.3064x over previous
import functools

import jax
import jax.numpy as jnp
from jax import lax
from jax.experimental import pallas as pl
from jax.experimental.pallas import tpu as pltpu
from jax.experimental.pallas import tpu_sc as plsc

N = 10000
E = 320000
D = 128

NC, NS = 2, 16
NW = NC * NS
CH = 128
CPW = 79
NCHUNK = NW * CPW
EPAD = NCHUNK * CH
ACC_ROWS = 10240
PAD_DST = N
RPW = ACC_ROWS // NS

MM_BLK = 1000


def _matmul(x, w):
    def body(x_ref, w_ref, o_ref):
        o_ref[...] = jnp.dot(x_ref[...], w_ref[...],
                             preferred_element_type=jnp.float32)

    return pl.pallas_call(
        body,
        grid=(N // MM_BLK,),
        in_specs=[
            pl.BlockSpec((MM_BLK, D), lambda i: (i, 0)),
            pl.BlockSpec((D, D), lambda i: (0, 0)),
        ],
        out_specs=pl.BlockSpec((MM_BLK, D), lambda i: (i, 0)),
        out_shape=jax.ShapeDtypeStruct((N, D), jnp.float32),
    )(x, w)


def _sc_scatter(h0, h1, h2, src, dst, zeros):
    mesh = plsc.VectorSubcoreMesh(core_axis_name="c", subcore_axis_name="s")

    @functools.partial(
        pl.kernel,
        out_type=jax.ShapeDtypeStruct((3, NC, ACC_ROWS, D), jnp.float32),
        mesh=mesh,
        scratch_types=[
            pltpu.VMEM((1, CH), jnp.int32),
            pltpu.VMEM((1, CH), jnp.int32),
            pltpu.VMEM((CH, D), jnp.float32),
            pltpu.VMEM_SHARED((ACC_ROWS, D), jnp.float32),
            pltpu.SemaphoreType.DMA,
        ],
    )
    def k(h0_hbm, h1_hbm, h2_hbm, src_hbm, dst_hbm, z_hbm, p_hbm,
          srcv, dstv, rows, acc, sem):
        c = lax.axis_index("c")
        s = lax.axis_index("s")
        wid = c * NS + s
        hs = (h0_hbm, h1_hbm, h2_hbm)

        pltpu.sync_copy(z_hbm.at[pl.ds(s * RPW, RPW)], acc.at[pl.ds(s * RPW, RPW)])
        plsc.subcore_barrier()

        for g in range(3):
            base = wid * CPW

            def body(i, carry, g=g):
                ci = base + i
                pltpu.sync_copy(src_hbm.at[g, ci], srcv.at[0])
                pltpu.sync_copy(dst_hbm.at[g, ci], dstv.at[0])
                pltpu.async_copy(hs[g].at[srcv.at[0]], rows, sem).wait()
                pltpu.sync_copy(rows, acc.at[dstv.at[0]], add=True)
                return carry

            lax.fori_loop(0, CPW, body, 0)
            plsc.subcore_barrier()
            pltpu.sync_copy(acc.at[pl.ds(s * RPW, RPW)],
                            p_hbm.at[g, c, pl.ds(s * RPW, RPW)])
            if g < 2:
                pltpu.sync_copy(z_hbm.at[pl.ds(s * RPW, RPW)],
                                acc.at[pl.ds(s * RPW, RPW)])
            plsc.subcore_barrier()

    return k(h0, h1, h2, src, dst, zeros)


def _epilogue(p, g, x, b):
    def body(p0_ref, p1_ref, x_ref, b_ref, o_ref):
        agg = p0_ref[0, 0] + p1_ref[0, 0] + b_ref[...]
        o_ref[...] = jnp.maximum(agg, 0.0) + x_ref[...]

    return pl.pallas_call(
        body,
        grid=(N // MM_BLK,),
        in_specs=[
            pl.BlockSpec((1, 1, MM_BLK, D), lambda i, g=g: (g, 0, i, 0)),
            pl.BlockSpec((1, 1, MM_BLK, D), lambda i, g=g: (g, 1, i, 0)),
            pl.BlockSpec((MM_BLK, D), lambda i: (i, 0)),
            pl.BlockSpec((1, D), lambda i: (0, 0)),
        ],
        out_specs=pl.BlockSpec((MM_BLK, D), lambda i: (i, 0)),
        out_shape=jax.ShapeDtypeStruct((N, D), jnp.float32),
    )(p, p, x, b)


def _prep_edges(edge_index):
    src = edge_index[0].astype(jnp.int32)
    dst = edge_index[1].astype(jnp.int32)
    src = jnp.concatenate([src, jnp.zeros((EPAD - E,), jnp.int32)])
    dst = jnp.concatenate([dst, jnp.full((EPAD - E,), PAD_DST, jnp.int32)])
    return src.reshape(NCHUNK, CH), dst.reshape(NCHUNK, CH)


def kernel(h_mf_new, h_bp_new, h_cc_new, mf_edge_index, bp_edge_index,
           cc_edge_index, W_mf, b_mf, W_bp, b_bp, W_cc, b_cc):
    xs = (h_mf_new, h_bp_new, h_cc_new)
    hs = tuple(_matmul(x, w) for x, w in zip(xs, (W_mf, W_bp, W_cc)))

    se, de = zip(*(_prep_edges(e) for e in
                   (mf_edge_index, bp_edge_index, cc_edge_index)))
    src = jnp.stack(se)
    dst = jnp.stack(de)
    zeros = jnp.zeros((ACC_ROWS, D), jnp.float32)

    p = _sc_scatter(hs[0], hs[1], hs[2], src, dst, zeros)

    bs = (b_mf, b_bp, b_cc)
    outs = tuple(_epilogue(p, g, xs[g], bs[g].reshape(1, D)) for g in range(3))
    return outs
